# Initial kernel scaffold; baseline (speedup 1.0000x reference)
import jax
import jax.numpy as jnp
from jax.experimental import pallas as pl


def kernel(x, edge_index, edge_attr, W1, b1, W2, b2, W3, b3):
    raise NotImplementedError("write your pallas kernel here")


# speedup vs baseline: 6.3258x; 6.3258x over previous
import functools

import jax
import jax.numpy as jnp
from jax import lax
from jax.experimental import pallas as pl
from jax.experimental.pallas import tpu as pltpu
from jax.experimental.pallas import tpu_sc as plsc

N = 10000
E = 320000
D = 128
DO = 64
DE = 16
NC = 2
NS = 16
NW = NC * NS
EPW = E // NW
CH = 80
NIT = EPW // CH
NP = 10240
NPS = NP // NS
RB = 1000
EB = 2000

_mesh = plsc.VectorSubcoreMesh(core_axis_name="c", subcore_axis_name="s")


def _sc_degree(dst, ones, zeros16):
    @functools.partial(
        pl.kernel, mesh=_mesh,
        out_type=jax.ShapeDtypeStruct((NC * NP, 16), jnp.float32),
        scratch_types=[pltpu.VMEM((CH,), jnp.int32),
                       pltpu.VMEM((CH, 16), jnp.float32),
                       pltpu.VMEM_SHARED((NP, 16), jnp.float32)],
        compiler_params=pltpu.CompilerParams(use_tc_tiling_on_sc=False),
    )
    def k(dst_hbm, ones_hbm, zeros_hbm, out_hbm, idx_v, ones_v, acc):
        cid = lax.axis_index("c")
        sid = lax.axis_index("s")
        base = (cid * NS + sid) * EPW
        pltpu.sync_copy(zeros_hbm.at[pl.ds(sid * NPS, NPS), :],
                        acc.at[pl.ds(sid * NPS, NPS), :])
        pltpu.sync_copy(ones_hbm, ones_v)
        plsc.subcore_barrier()

        @pl.loop(0, NIT)
        def _(i):
            pltpu.sync_copy(dst_hbm.at[pl.ds(base + i * CH, CH)], idx_v)
            pltpu.sync_copy(ones_v, acc.at[idx_v], add=True)

        plsc.subcore_barrier()
        pltpu.sync_copy(acc.at[pl.ds(sid * NPS, NPS), :],
                        out_hbm.at[pl.ds(cid * NP + sid * NPS, NPS), :])

    return k(dst, ones, zeros16)


def _sc_scatter(y, src, dst, zeros):
    @functools.partial(
        pl.kernel, mesh=_mesh,
        out_type=jax.ShapeDtypeStruct((NC * NP, D), jnp.float32),
        scratch_types=[pltpu.VMEM((CH,), jnp.int32),
                       pltpu.VMEM((CH,), jnp.int32),
                       pltpu.VMEM((CH, D), jnp.float32),
                       pltpu.VMEM_SHARED((NP, D), jnp.float32)],
    )
    def k(y_hbm, src_hbm, dst_hbm, zeros_hbm, out_hbm, si_v, di_v, rows_v, acc):
        cid = lax.axis_index("c")
        sid = lax.axis_index("s")
        base = (cid * NS + sid) * EPW
        pltpu.sync_copy(zeros_hbm.at[pl.ds(sid * NPS, NPS), :],
                        acc.at[pl.ds(sid * NPS, NPS), :])
        plsc.subcore_barrier()

        @pl.loop(0, NIT)
        def _(i):
            pltpu.sync_copy(src_hbm.at[pl.ds(base + i * CH, CH)], si_v)
            pltpu.sync_copy(dst_hbm.at[pl.ds(base + i * CH, CH)], di_v)
            pltpu.sync_copy(y_hbm.at[si_v], rows_v)
            pltpu.sync_copy(rows_v, acc.at[di_v], add=True)

        plsc.subcore_barrier()
        pltpu.sync_copy(acc.at[pl.ds(sid * NPS, NPS), :],
                        out_hbm.at[pl.ds(cid * NP + sid * NPS, NPS), :])

    return k(y, src, dst, zeros)


def _sc_edge_gather(P, Q, src, dst):
    out_t = (jax.ShapeDtypeStruct((E, DO), jnp.float32),
             jax.ShapeDtypeStruct((E, DO), jnp.float32))

    @functools.partial(
        pl.kernel, mesh=_mesh,
        out_type=out_t,
        scratch_types=[pltpu.VMEM((CH,), jnp.int32),
                       pltpu.VMEM((CH,), jnp.int32),
                       pltpu.VMEM((CH, DO), jnp.float32),
                       pltpu.VMEM((CH, DO), jnp.float32)],
        compiler_params=pltpu.CompilerParams(use_tc_tiling_on_sc=False),
    )
    def k(p_hbm, q_hbm, src_hbm, dst_hbm, ps_hbm, qd_hbm, si_v, di_v, pr_v, qr_v):
        cid = lax.axis_index("c")
        sid = lax.axis_index("s")
        base = (cid * NS + sid) * EPW

        @pl.loop(0, NIT)
        def _(i):
            off = base + i * CH
            pltpu.sync_copy(src_hbm.at[pl.ds(off, CH)], si_v)
            pltpu.sync_copy(dst_hbm.at[pl.ds(off, CH)], di_v)
            pltpu.sync_copy(p_hbm.at[si_v], pr_v)
            pltpu.sync_copy(q_hbm.at[di_v], qr_v)
            pltpu.sync_copy(pr_v, ps_hbm.at[pl.ds(off, CH), :])
            pltpu.sync_copy(qr_v, qd_hbm.at[pl.ds(off, CH), :])

    return k(P, Q, src, dst)


def _tc_layer0(degp, x, W1):
    def body(dp_ref, x_ref, w_ref, y_ref, dinv_ref):
        dp = dp_ref[...]
        deg = dp[0, :, 0:1] + dp[1, :, 0:1] + 1.0
        dinv = lax.rsqrt(deg)
        dinv_ref[...] = dinv
        y_ref[...] = jnp.dot(x_ref[...], w_ref[...],
                             preferred_element_type=jnp.float32) * dinv

    return pl.pallas_call(
        body,
        grid=(N // RB,),
        in_specs=[pl.BlockSpec((NC, RB, 16), lambda i: (0, i, 0)),
                  pl.BlockSpec((RB, D), lambda i: (i, 0)),
                  pl.BlockSpec((D, D), lambda i: (0, 0))],
        out_specs=[pl.BlockSpec((RB, D), lambda i: (i, 0)),
                   pl.BlockSpec((RB, 1), lambda i: (i, 0))],
        out_shape=[jax.ShapeDtypeStruct((N, D), jnp.float32),
                   jax.ShapeDtypeStruct((N, 1), jnp.float32)],
    )(degp, x, W1)


def _tc_mid(part, y, dinv, W, b):
    def body(p_ref, y_ref, dinv_ref, w_ref, b_ref, o_ref):
        p = p_ref[...]
        dinv = dinv_ref[...]
        h = jnp.maximum(dinv * (p[0] + p[1] + y_ref[...]) + b_ref[...], 0.0)
        o_ref[...] = jnp.dot(h, w_ref[...],
                             preferred_element_type=jnp.float32) * dinv

    return pl.pallas_call(
        body,
        grid=(N // RB,),
        in_specs=[pl.BlockSpec((NC, RB, D), lambda i: (0, i, 0)),
                  pl.BlockSpec((RB, D), lambda i: (i, 0)),
                  pl.BlockSpec((RB, 1), lambda i: (i, 0)),
                  pl.BlockSpec((D, D), lambda i: (0, 0)),
                  pl.BlockSpec((1, D), lambda i: (0, 0))],
        out_specs=pl.BlockSpec((RB, D), lambda i: (i, 0)),
        out_shape=jax.ShapeDtypeStruct((N, D), jnp.float32),
    )(part, y, dinv, W, b)


def _tc_last_nodes(part, y, dinv, W3a, W3b, b):
    def body(p_ref, y_ref, dinv_ref, wa_ref, wb_ref, b_ref, P_ref, Q_ref):
        p = p_ref[...]
        dinv = dinv_ref[...]
        h = jnp.maximum(dinv * (p[0] + p[1] + y_ref[...]) + b_ref[...], 0.0)
        P_ref[...] = jnp.dot(h, wa_ref[...], preferred_element_type=jnp.float32)
        Q_ref[...] = jnp.dot(h, wb_ref[...], preferred_element_type=jnp.float32)

    return pl.pallas_call(
        body,
        grid=(N // RB,),
        in_specs=[pl.BlockSpec((NC, RB, D), lambda i: (0, i, 0)),
                  pl.BlockSpec((RB, D), lambda i: (i, 0)),
                  pl.BlockSpec((RB, 1), lambda i: (i, 0)),
                  pl.BlockSpec((D, DO), lambda i: (0, 0)),
                  pl.BlockSpec((D, DO), lambda i: (0, 0)),
                  pl.BlockSpec((1, D), lambda i: (0, 0))],
        out_specs=[pl.BlockSpec((RB, DO), lambda i: (i, 0)),
                   pl.BlockSpec((RB, DO), lambda i: (i, 0))],
        out_shape=[jax.ShapeDtypeStruct((N, DO), jnp.float32),
                   jax.ShapeDtypeStruct((N, DO), jnp.float32)],
    )(part, y, dinv, W3a, W3b, b)


def _tc_edges(ps, qd, ea, W3c, b3):
    def body(ps_ref, qd_ref, ea_ref, w_ref, b_ref, o_ref):
        o_ref[...] = (ps_ref[...] + qd_ref[...]
                      + jnp.dot(ea_ref[...], w_ref[...],
                                preferred_element_type=jnp.float32)
                      + b_ref[...])

    return pl.pallas_call(
        body,
        grid=(E // EB,),
        in_specs=[pl.BlockSpec((EB, DO), lambda i: (i, 0)),
                  pl.BlockSpec((EB, DO), lambda i: (i, 0)),
                  pl.BlockSpec((EB, DE), lambda i: (i, 0)),
                  pl.BlockSpec((DE, DO), lambda i: (0, 0)),
                  pl.BlockSpec((1, DO), lambda i: (0, 0))],
        out_specs=pl.BlockSpec((EB, DO), lambda i: (i, 0)),
        out_shape=jax.ShapeDtypeStruct((E, DO), jnp.float32),
    )(ps, qd, ea, W3c, b3)


def kernel(x, edge_index, edge_attr, W1, b1, W2, b2, W3, b3):
    src = edge_index[0]
    dst = edge_index[1]
    zeros_d = jnp.zeros((NP, D), jnp.float32)
    zeros16 = jnp.zeros((NP, 16), jnp.float32)
    ones_ch = jnp.ones((CH, 16), jnp.float32)

    degp = _sc_degree(dst, ones_ch, zeros16).reshape(NC, NP, 16)
    y1, dinv = _tc_layer0(degp, x, W1)
    part1 = _sc_scatter(y1, src, dst, zeros_d).reshape(NC, NP, D)
    y2 = _tc_mid(part1, y1, dinv, W2, b1.reshape(1, D))
    part2 = _sc_scatter(y2, src, dst, zeros_d).reshape(NC, NP, D)
    P, Q = _tc_last_nodes(part2, y2, dinv, W3[:D], W3[D:2 * D], b2.reshape(1, D))
    ps, qd = _sc_edge_gather(P, Q, src, dst)
    return _tc_edges(ps, qd, edge_attr, W3[2 * D:], b3.reshape(1, DO))


# speedup vs baseline: 6.7148x; 1.0615x over previous
import functools

import jax
import jax.numpy as jnp
from jax import lax
from jax.experimental import pallas as pl
from jax.experimental.pallas import tpu as pltpu
from jax.experimental.pallas import tpu_sc as plsc

N = 10000
E = 320000
D = 128
DO = 64
DE = 16
NC = 2
NS = 16
NW = NC * NS
EPW = E // NW
CH = 80
NIT = EPW // CH
NP = 10240
NPS = NP // NS
RB = 1000
EB = 2000

_mesh = plsc.VectorSubcoreMesh(core_axis_name="c", subcore_axis_name="s")


def _sc_degree(dst, ones, zeros16):
    @functools.partial(
        pl.kernel, mesh=_mesh,
        out_type=jax.ShapeDtypeStruct((NC * NP, 16), jnp.float32),
        scratch_types=[pltpu.VMEM((CH,), jnp.int32),
                       pltpu.VMEM((CH, 16), jnp.float32),
                       pltpu.VMEM_SHARED((NP, 16), jnp.float32)],
        compiler_params=pltpu.CompilerParams(use_tc_tiling_on_sc=False),
    )
    def k(dst_hbm, ones_hbm, zeros_hbm, out_hbm, idx_v, ones_v, acc):
        cid = lax.axis_index("c")
        sid = lax.axis_index("s")
        base = (cid * NS + sid) * EPW
        pltpu.sync_copy(zeros_hbm.at[pl.ds(sid * NPS, NPS), :],
                        acc.at[pl.ds(sid * NPS, NPS), :])
        pltpu.sync_copy(ones_hbm, ones_v)
        plsc.subcore_barrier()

        @pl.loop(0, NIT)
        def _(i):
            pltpu.sync_copy(dst_hbm.at[pl.ds(base + i * CH, CH)], idx_v)
            pltpu.sync_copy(ones_v, acc.at[idx_v], add=True)

        plsc.subcore_barrier()
        pltpu.sync_copy(acc.at[pl.ds(sid * NPS, NPS), :],
                        out_hbm.at[pl.ds(cid * NP + sid * NPS, NPS), :])

    return k(dst, ones, zeros16)


def _sc_scatter(y, src, dst, zeros):
    @functools.partial(
        pl.kernel, mesh=_mesh,
        out_type=jax.ShapeDtypeStruct((NC * NP, D), jnp.float32),
        scratch_types=[pltpu.VMEM((CH,), jnp.int32),
                       pltpu.VMEM((CH,), jnp.int32),
                       pltpu.VMEM((CH, D), jnp.float32),
                       pltpu.VMEM_SHARED((NP, D), jnp.float32)],
    )
    def k(y_hbm, src_hbm, dst_hbm, zeros_hbm, out_hbm, si_v, di_v, rows_v, acc):
        cid = lax.axis_index("c")
        sid = lax.axis_index("s")
        base = (cid * NS + sid) * EPW
        pltpu.sync_copy(zeros_hbm.at[pl.ds(sid * NPS, NPS), :],
                        acc.at[pl.ds(sid * NPS, NPS), :])
        plsc.subcore_barrier()

        @pl.loop(0, NIT)
        def _(i):
            pltpu.sync_copy(src_hbm.at[pl.ds(base + i * CH, CH)], si_v)
            pltpu.sync_copy(dst_hbm.at[pl.ds(base + i * CH, CH)], di_v)
            pltpu.sync_copy(y_hbm.at[si_v], rows_v)
            pltpu.sync_copy(rows_v, acc.at[di_v], add=True)

        plsc.subcore_barrier()
        pltpu.sync_copy(acc.at[pl.ds(sid * NPS, NPS), :],
                        out_hbm.at[pl.ds(cid * NP + sid * NPS, NPS), :])

    return k(y, src, dst, zeros)


def _sc_edge_combine(T1, T2, src, dst):
    @functools.partial(
        pl.kernel, mesh=_mesh,
        out_type=jax.ShapeDtypeStruct((E, D), jnp.float32),
        scratch_types=[pltpu.VMEM((CH,), jnp.int32),
                       pltpu.VMEM((CH,), jnp.int32),
                       pltpu.VMEM((CH,), jnp.int32),
                       pltpu.VMEM((CH, D), jnp.float32),
                       pltpu.VMEM((CH, D), jnp.float32),
                       pltpu.VMEM_SHARED((NS * CH, D), jnp.float32)],
    )
    def k(t1_hbm, t2_hbm, src_hbm, dst_hbm, s_hbm, si_v, di_v, id_v, a_v, b_v, sp):
        cid = lax.axis_index("c")
        sid = lax.axis_index("s")
        base = (cid * NS + sid) * EPW
        for j in range(CH // 16):
            id_v[pl.ds(16 * j, 16)] = lax.iota(jnp.int32, 16) + (sid * CH + 16 * j)

        @pl.loop(0, NIT)
        def _(i):
            off = base + i * CH
            pltpu.sync_copy(src_hbm.at[pl.ds(off, CH)], si_v)
            pltpu.sync_copy(dst_hbm.at[pl.ds(off, CH)], di_v)
            pltpu.sync_copy(t1_hbm.at[si_v], a_v)
            pltpu.sync_copy(t2_hbm.at[di_v], b_v)
            pltpu.sync_copy(a_v, sp.at[pl.ds(sid * CH, CH), :])
            pltpu.sync_copy(b_v, sp.at[id_v], add=True)
            pltpu.sync_copy(sp.at[pl.ds(sid * CH, CH), :], s_hbm.at[pl.ds(off, CH), :])

    return k(T1, T2, src, dst)


def _tc_layer0(degp, x, W1):
    def body(dp_ref, x_ref, w_ref, y_ref, dinv_ref):
        dp = dp_ref[...]
        deg = dp[0, :, 0:1] + dp[1, :, 0:1] + 1.0
        dinv = lax.rsqrt(deg)
        dinv_ref[...] = dinv
        y_ref[...] = jnp.dot(x_ref[...], w_ref[...],
                             preferred_element_type=jnp.float32) * dinv

    return pl.pallas_call(
        body,
        grid=(N // RB,),
        in_specs=[pl.BlockSpec((NC, RB, 16), lambda i: (0, i, 0)),
                  pl.BlockSpec((RB, D), lambda i: (i, 0)),
                  pl.BlockSpec((D, D), lambda i: (0, 0))],
        out_specs=[pl.BlockSpec((RB, D), lambda i: (i, 0)),
                   pl.BlockSpec((RB, 1), lambda i: (i, 0))],
        out_shape=[jax.ShapeDtypeStruct((N, D), jnp.float32),
                   jax.ShapeDtypeStruct((N, 1), jnp.float32)],
    )(degp, x, W1)


def _tc_mid(part, y, dinv, W, b):
    def body(p_ref, y_ref, dinv_ref, w_ref, b_ref, o_ref):
        p = p_ref[...]
        dinv = dinv_ref[...]
        h = jnp.maximum(dinv * (p[0] + p[1] + y_ref[...]) + b_ref[...], 0.0)
        o_ref[...] = jnp.dot(h, w_ref[...],
                             preferred_element_type=jnp.float32) * dinv

    return pl.pallas_call(
        body,
        grid=(N // RB,),
        in_specs=[pl.BlockSpec((NC, RB, D), lambda i: (0, i, 0)),
                  pl.BlockSpec((RB, D), lambda i: (i, 0)),
                  pl.BlockSpec((RB, 1), lambda i: (i, 0)),
                  pl.BlockSpec((D, D), lambda i: (0, 0)),
                  pl.BlockSpec((1, D), lambda i: (0, 0))],
        out_specs=pl.BlockSpec((RB, D), lambda i: (i, 0)),
        out_shape=jax.ShapeDtypeStruct((N, D), jnp.float32),
    )(part, y, dinv, W, b)


def _tc_last_nodes(part, y, dinv, W12, W21, b):
    def body(p_ref, y_ref, dinv_ref, wa_ref, wb_ref, b_ref, T1_ref, T2_ref):
        p = p_ref[...]
        dinv = dinv_ref[...]
        h = jnp.maximum(dinv * (p[0] + p[1] + y_ref[...]) + b_ref[...], 0.0)
        T1_ref[...] = jnp.dot(h, wa_ref[...], preferred_element_type=jnp.float32)
        T2_ref[...] = jnp.dot(h, wb_ref[...], preferred_element_type=jnp.float32)

    return pl.pallas_call(
        body,
        grid=(N // RB,),
        in_specs=[pl.BlockSpec((NC, RB, D), lambda i: (0, i, 0)),
                  pl.BlockSpec((RB, D), lambda i: (i, 0)),
                  pl.BlockSpec((RB, 1), lambda i: (i, 0)),
                  pl.BlockSpec((D, D), lambda i: (0, 0)),
                  pl.BlockSpec((D, D), lambda i: (0, 0)),
                  pl.BlockSpec((1, D), lambda i: (0, 0))],
        out_specs=[pl.BlockSpec((RB, D), lambda i: (i, 0)),
                   pl.BlockSpec((RB, D), lambda i: (i, 0))],
        out_shape=[jax.ShapeDtypeStruct((N, D), jnp.float32),
                   jax.ShapeDtypeStruct((N, D), jnp.float32)],
    )(part, y, dinv, W12, W21, b)


def _tc_edges(S, ea, W3c, b3):
    def body(s_ref, ea_ref, w_ref, b_ref, o_ref):
        o_ref[...] = (s_ref[:, :DO]
                      + jnp.dot(ea_ref[...], w_ref[...],
                                preferred_element_type=jnp.float32)
                      + b_ref[...])

    return pl.pallas_call(
        body,
        grid=(E // EB,),
        in_specs=[pl.BlockSpec((EB, D), lambda i: (i, 0)),
                  pl.BlockSpec((EB, DE), lambda i: (i, 0)),
                  pl.BlockSpec((DE, DO), lambda i: (0, 0)),
                  pl.BlockSpec((1, DO), lambda i: (0, 0))],
        out_specs=pl.BlockSpec((EB, DO), lambda i: (i, 0)),
        out_shape=jax.ShapeDtypeStruct((E, DO), jnp.float32),
    )(S, ea, W3c, b3)


def kernel(x, edge_index, edge_attr, W1, b1, W2, b2, W3, b3):
    src = edge_index[0]
    dst = edge_index[1]
    zeros_d = jnp.zeros((NP, D), jnp.float32)
    zeros16 = jnp.zeros((NP, 16), jnp.float32)
    ones_ch = jnp.ones((CH, 16), jnp.float32)

    degp = _sc_degree(dst, ones_ch, zeros16).reshape(NC, NP, 16)
    y1, dinv = _tc_layer0(degp, x, W1)
    part1 = _sc_scatter(y1, src, dst, zeros_d).reshape(NC, NP, D)
    y2 = _tc_mid(part1, y1, dinv, W2, b1.reshape(1, D))
    part2 = _sc_scatter(y2, src, dst, zeros_d).reshape(NC, NP, D)
    W3a, W3b = W3[:D], W3[D:2 * D]
    W12 = jnp.concatenate([W3a, W3b], axis=1)
    W21 = jnp.concatenate([W3b, W3a], axis=1)
    T1, T2 = _tc_last_nodes(part2, y2, dinv, W12, W21, b2.reshape(1, D))
    S = _sc_edge_combine(T1, T2, src, dst)
    return _tc_edges(S, edge_attr, W3[2 * D:], b3.reshape(1, DO))
